# Initial kernel scaffold; baseline (speedup 1.0000x reference)
import jax
import jax.numpy as jnp
from jax.experimental import pallas as pl


def kernel(node_ids, edge_src, edge_dst, gnn_ind, batch_gnn_ind, W_emb, W_iou, U_iou, b_iou, W_f, U_f, b_f, W_out, R1, R1b, R2, R2b, V1, V1b, V2, V2b, Q1, Q1b, Q2, Q2b, L, Lb):
    raise NotImplementedError("write your pallas kernel here")


# speedup vs baseline: 1.0297x; 1.0297x over previous
import functools

import jax
import jax.numpy as jnp
from jax import lax
from jax.experimental import pallas as pl
from jax.experimental.pallas import tpu as pltpu

N = 10000
E = 320000
H = 128
NUM_IN = 2004
G = 256
B = 32
NUM_LEMMAS = 1000

BLK1 = 2000
BLK2 = 2000


def _sig(x):
    return jax.nn.sigmoid(x)


def _k1_body(ids_ref, W_emb_ref, W_iou_ref, b_iou_ref, W_f_ref, b_f_ref,
             U_f_ref, x_ioub_ref, h1_ref, pack_ref, mxfb_ref):
    ids = ids_ref[...]
    onehot = (ids == lax.broadcasted_iota(jnp.int32, (BLK1, NUM_IN), 1))
    x_emb = onehot.astype(jnp.float32) @ W_emb_ref[...]
    x_ioub = x_emb @ W_iou_ref[...] + b_iou_ref[...]
    i1 = x_ioub[:, :H]
    o1 = x_ioub[:, H:2 * H]
    u1 = x_ioub[:, 2 * H:]
    c1 = _sig(i1) * jnp.tanh(u1)
    h1 = _sig(o1) * jnp.tanh(c1)
    x_ioub_ref[...] = x_ioub
    h1_ref[...] = h1
    pack_ref[...] = jnp.concatenate([c1, h1 @ U_f_ref[...]], axis=1)
    mxfb_ref[...] = -(x_emb @ W_f_ref[...] + b_f_ref[...])


def _run_k1(ids2d, W_emb, W_iou, b_iou, W_f, b_f, U_f):
    n_blocks = N // BLK1
    full = lambda shape: pl.BlockSpec(shape, lambda i: (0,) * len(shape))
    return pl.pallas_call(
        _k1_body,
        grid=(n_blocks,),
        in_specs=[
            pl.BlockSpec((BLK1, 1), lambda i: (i, 0)),
            full((NUM_IN, H)),
            full((H, 3 * H)),
            full((1, 3 * H)),
            full((H, H)),
            full((1, H)),
            full((H, H)),
        ],
        out_specs=[
            pl.BlockSpec((BLK1, 3 * H), lambda i: (i, 0)),
            pl.BlockSpec((BLK1, H), lambda i: (i, 0)),
            pl.BlockSpec((BLK1, 2 * H), lambda i: (i, 0)),
            pl.BlockSpec((BLK1, H), lambda i: (i, 0)),
        ],
        out_shape=[
            jax.ShapeDtypeStruct((N, 3 * H), jnp.float32),
            jax.ShapeDtypeStruct((N, H), jnp.float32),
            jax.ShapeDtypeStruct((N, 2 * H), jnp.float32),
            jax.ShapeDtypeStruct((N, H), jnp.float32),
        ],
    )(ids2d, W_emb, W_iou, b_iou, W_f, b_f, U_f)


def _edge_stage(h1, pack, mxfb, gather_idx, scatter_idx):
    c1 = pack[:, :H]
    hU = pack[:, H:]
    h_src = h1[gather_idx]
    h_sum = jnp.zeros_like(h1).at[scatter_idx].add(h_src)
    t = mxfb[scatter_idx] - hU[gather_idx]
    val = c1[gather_idx] / (1.0 + jnp.exp(t))
    fc = jnp.zeros_like(h1).at[scatter_idx].add(val)
    return h_sum, fc


def _k2_body(x_ioub_ref, hsf_ref, fcf_ref, hsb_ref, fcb_ref, U_iou_ref,
             W_out_ref, R1_ref, R1b_ref, R2_ref, R2b_ref, gnn_ref, bgi_ref,
             V1_ref, V1b_ref, V2_ref, V2b_ref, Q1_ref, Q1b_ref, Q2_ref,
             Q2b_ref, L_ref, Lb_ref, out_ref, outg_ref):
    step = pl.program_id(0)
    nsteps = pl.num_programs(0)

    def direction(hs, fc):
        iou = x_ioub_ref[...] + hs @ U_iou_ref[...]
        i = iou[:, :H]
        o = iou[:, H:2 * H]
        u = iou[:, 2 * H:]
        c = _sig(i) * jnp.tanh(u) + fc
        h = _sig(o) * jnp.tanh(c)
        return h @ W_out_ref[...]

    fwd = direction(hsf_ref[...], fcf_ref[...])
    bwd = direction(hsb_ref[...], fcb_ref[...])
    state = jnp.concatenate([fwd, bwd], axis=1)
    state = jax.nn.relu(
        state + jax.nn.relu(state @ R1_ref[...] + R1b_ref[...]) @ R2_ref[...]
        + R2b_ref[...])

    oh = (gnn_ref[...] == lax.broadcasted_iota(jnp.int32, (BLK2, G), 1))
    contrib = lax.dot_general(oh.astype(jnp.float32), state,
                              (((0,), (0,)), ((), ())))

    @pl.when(step == 0)
    def _():
        outg_ref[...] = contrib

    @pl.when(step != 0)
    def _():
        outg_ref[...] = outg_ref[...] + contrib

    @pl.when(step == nsteps - 1)
    def _():
        oh2 = (bgi_ref[...] == lax.broadcasted_iota(jnp.int32, (G, B), 1))
        obj = lax.dot_general(oh2.astype(jnp.float32), outg_ref[...],
                              (((0,), (0,)), ((), ())))
        out = jnp.concatenate([obj, jnp.zeros_like(obj)], axis=1)
        vf = jax.nn.relu(out @ V1_ref[...] + V1b_ref[...]) @ V2_ref[...] \
            + V2b_ref[...]
        lemma_q = jax.nn.relu(
            out + jax.nn.relu(out @ Q1_ref[...] + Q1b_ref[...]) @ Q2_ref[...]
            + Q2b_ref[...])
        lemma = jax.nn.relu(lemma_q) @ L_ref[...] + Lb_ref[...]
        out_ref[...] = jnp.concatenate([vf, lemma], axis=1)


def _run_k2(x_ioub, hsf, fcf, hsb, fcb, U_iou, W_out, R1, R1b, R2, R2b,
            gnn2d, bgi2d, V1, V1b, V2, V2b, Q1, Q1b, Q2, Q2b, L, Lb):
    n_blocks = N // BLK2
    full = lambda shape: pl.BlockSpec(shape, lambda i: (0,) * len(shape))
    row = lambda w: pl.BlockSpec((BLK2, w), lambda i: (i, 0))
    return pl.pallas_call(
        _k2_body,
        grid=(n_blocks,),
        in_specs=[
            row(3 * H), row(H), row(H), row(H), row(H),
            full((H, 3 * H)),
            full((H, 64)),
            full((2 * 64, 2 * 64)), full((1, 2 * 64)),
            full((2 * 64, 2 * 64)), full((1, 2 * 64)),
            pl.BlockSpec((BLK2, 1), lambda i: (i, 0)),
            full((G, 1)),
            full((2 * H, H)), full((1, H)),
            full((H, 1)), full((1, 1)),
            full((2 * H, 2 * H)), full((1, 2 * H)),
            full((2 * H, 2 * H)), full((1, 2 * H)),
            full((2 * H, NUM_LEMMAS)), full((1, NUM_LEMMAS)),
        ],
        out_specs=pl.BlockSpec((B, 1 + NUM_LEMMAS), lambda i: (0, 0)),
        out_shape=jax.ShapeDtypeStruct((B, 1 + NUM_LEMMAS), jnp.float32),
        scratch_shapes=[pltpu.VMEM((G, H), jnp.float32)],
    )(x_ioub, hsf, fcf, hsb, fcb, U_iou, W_out, R1, R1b, R2, R2b, gnn2d,
      bgi2d, V1, V1b, V2, V2b, Q1, Q1b, Q2, Q2b, L, Lb)


def kernel(node_ids, edge_src, edge_dst, gnn_ind, batch_gnn_ind, W_emb,
           W_iou, U_iou, b_iou, W_f, U_f, b_f, W_out, R1, R1b, R2, R2b, V1,
           V1b, V2, V2b, Q1, Q1b, Q2, Q2b, L, Lb):
    i32 = jnp.int32
    ids2d = node_ids.astype(i32).reshape(N, 1)
    gnn2d = gnn_ind.astype(i32).reshape(N, 1)
    bgi2d = batch_gnn_ind.astype(i32).reshape(G, 1)
    row1 = lambda v: v.reshape(1, -1)

    x_ioub, h1, pack, mxfb = _run_k1(
        ids2d, W_emb, W_iou, row1(b_iou), W_f, row1(b_f), U_f)

    hsf, fcf = _edge_stage(h1, pack, mxfb, edge_src, edge_dst)
    hsb, fcb = _edge_stage(h1, pack, mxfb, edge_dst, edge_src)

    return _run_k2(x_ioub, hsf, fcf, hsb, fcb, U_iou, W_out, R1, row1(R1b),
                   R2, row1(R2b), gnn2d, bgi2d, V1, row1(V1b), V2, row1(V2b),
                   Q1, row1(Q1b), Q2, row1(Q2b), L, row1(Lb))


# speedup vs baseline: 1.4970x; 1.4538x over previous
import functools

import jax
import jax.numpy as jnp
from jax import lax
from jax.experimental import pallas as pl
from jax.experimental.pallas import tpu as pltpu
from jax.experimental.pallas import tpu_sc as plsc

N = 10000
E = 320000
H = 128
NUM_IN = 2004
G = 256
B = 32
NUM_LEMMAS = 1000

BLK1 = 2000
BLK2 = 2000


def _sig(x):
    return jax.nn.sigmoid(x)


def _k1_body(ids_ref, W_emb_ref, W_iou_ref, b_iou_ref, W_f_ref, b_f_ref,
             U_f_ref, x_ioub_ref, h1_ref, pack_ref, mxfb_ref):
    ids = ids_ref[...]
    onehot = (ids == lax.broadcasted_iota(jnp.int32, (BLK1, NUM_IN), 1))
    x_emb = onehot.astype(jnp.float32) @ W_emb_ref[...]
    x_ioub = x_emb @ W_iou_ref[...] + b_iou_ref[...]
    i1 = x_ioub[:, :H]
    o1 = x_ioub[:, H:2 * H]
    u1 = x_ioub[:, 2 * H:]
    c1 = _sig(i1) * jnp.tanh(u1)
    h1 = _sig(o1) * jnp.tanh(c1)
    x_ioub_ref[...] = x_ioub
    h1_ref[...] = h1
    pack_ref[...] = jnp.concatenate([c1, h1 @ U_f_ref[...]], axis=1)
    mxfb_ref[...] = -(x_emb @ W_f_ref[...] + b_f_ref[...])


def _run_k1(ids2d, W_emb, W_iou, b_iou, W_f, b_f, U_f):
    n_blocks = N // BLK1
    full = lambda shape: pl.BlockSpec(shape, lambda i: (0,) * len(shape))
    return pl.pallas_call(
        _k1_body,
        grid=(n_blocks,),
        in_specs=[
            pl.BlockSpec((BLK1, 1), lambda i: (i, 0)),
            full((NUM_IN, H)),
            full((H, 3 * H)),
            full((1, 3 * H)),
            full((H, H)),
            full((1, H)),
            full((H, H)),
        ],
        out_specs=[
            pl.BlockSpec((BLK1, 3 * H), lambda i: (i, 0)),
            pl.BlockSpec((BLK1, H), lambda i: (i, 0)),
            pl.BlockSpec((BLK1, 2 * H), lambda i: (i, 0)),
            pl.BlockSpec((BLK1, H), lambda i: (i, 0)),
        ],
        out_shape=[
            jax.ShapeDtypeStruct((N, 3 * H), jnp.float32),
            jax.ShapeDtypeStruct((N, H), jnp.float32),
            jax.ShapeDtypeStruct((N, 2 * H), jnp.float32),
            jax.ShapeDtypeStruct((N, H), jnp.float32),
        ],
    )(ids2d, W_emb, W_iou, b_iou, W_f, b_f, U_f)


KE = 80
TILES = 16
EPT = E // TILES
CHUNKS = EPT // KE
RPT = 624


def _edge_sc_body(gidx_hbm, sidx_hbm, h1_hbm, pack_hbm, mxfb_hbm, hsum_hbm,
                  fc_hbm, acc, gidx, sidx, prows, mrows, val, sem):
    c = lax.axis_index("c")
    s = lax.axis_index("s")
    zero16 = jnp.zeros((16,), jnp.float32)

    @pl.loop(0, KE)
    def _(r):
        for j in range(8):
            val[r, pl.ds(16 * j, 16)] = zero16

    def zero_acc():
        for j in range(7):
            pltpu.sync_copy(val, acc.at[pl.ds(s * RPT + j * KE, KE)])
        pltpu.sync_copy(val.at[pl.ds(0, 64)],
                        acc.at[pl.ds(s * RPT + 7 * KE, 64)])

        @pl.when(s == TILES - 1)
        def _():
            pltpu.sync_copy(val.at[pl.ds(0, 16)],
                            acc.at[pl.ds(TILES * RPT, N - TILES * RPT)])

    def load_idx(i):
        base = c * E + s * EPT + i * KE
        pltpu.sync_copy(gidx_hbm.at[pl.ds(base, KE)], gidx)
        pltpu.sync_copy(sidx_hbm.at[pl.ds(base, KE)], sidx)

    def drain(out_hbm):
        pltpu.sync_copy(acc.at[pl.ds(s * RPT, RPT)],
                        out_hbm.at[c, pl.ds(s * RPT, RPT)])

        @pl.when(s == TILES - 1)
        def _():
            rem = N - TILES * RPT
            pltpu.sync_copy(acc.at[pl.ds(TILES * RPT, rem)],
                            out_hbm.at[c, pl.ds(TILES * RPT, rem)])

    zero_acc()
    plsc.subcore_barrier()

    @pl.loop(0, CHUNKS)
    def _(i):
        load_idx(i)
        pltpu.async_copy(h1_hbm.at[gidx], mrows, sem).wait()
        pltpu.sync_copy(mrows, acc.at[sidx], add=True)

    plsc.subcore_barrier()
    drain(hsum_hbm)
    zero_acc()
    plsc.subcore_barrier()

    @pl.loop(0, CHUNKS)
    def _(i):
        load_idx(i)
        pltpu.async_copy(pack_hbm.at[gidx], prows, sem).wait()
        pltpu.async_copy(mxfb_hbm.at[sidx], mrows, sem).wait()

        @pl.loop(0, KE)
        def _(e):
            for j in range(8):
                c1v = prows[e, pl.ds(16 * j, 16)]
                hUv = prows[e, pl.ds(H + 16 * j, 16)]
                mv = mrows[e, pl.ds(16 * j, 16)]
                val[e, pl.ds(16 * j, 16)] = c1v / (1.0 + jnp.exp(mv - hUv))

        pltpu.sync_copy(val, acc.at[sidx], add=True)

    plsc.subcore_barrier()
    drain(fc_hbm)


def _run_edge_sc(gidx_all, sidx_all, h1, pack, mxfb):
    mesh = plsc.VectorSubcoreMesh(core_axis_name="c", subcore_axis_name="s")
    f32 = jnp.float32
    return pl.kernel(
        _edge_sc_body,
        out_type=[
            jax.ShapeDtypeStruct((2, N, H), f32),
            jax.ShapeDtypeStruct((2, N, H), f32),
        ],
        mesh=mesh,
        scratch_types=[
            pltpu.VMEM_SHARED((N, H), f32),
            pltpu.VMEM((KE,), jnp.int32),
            pltpu.VMEM((KE,), jnp.int32),
            pltpu.VMEM((KE, 2 * H), f32),
            pltpu.VMEM((KE, H), f32),
            pltpu.VMEM((KE, H), f32),
            pltpu.SemaphoreType.DMA,
        ],
    )(gidx_all, sidx_all, h1, pack, mxfb)


def _k2_body(x_ioub_ref, hsf_ref, fcf_ref, hsb_ref, fcb_ref, U_iou_ref,
             W_out_ref, R1_ref, R1b_ref, R2_ref, R2b_ref, gnn_ref, bgi_ref,
             V1_ref, V1b_ref, V2_ref, V2b_ref, Q1_ref, Q1b_ref, Q2_ref,
             Q2b_ref, L_ref, Lb_ref, out_ref, outg_ref):
    step = pl.program_id(0)
    nsteps = pl.num_programs(0)

    def direction(hs, fc):
        iou = x_ioub_ref[...] + hs @ U_iou_ref[...]
        i = iou[:, :H]
        o = iou[:, H:2 * H]
        u = iou[:, 2 * H:]
        c = _sig(i) * jnp.tanh(u) + fc
        h = _sig(o) * jnp.tanh(c)
        return h @ W_out_ref[...]

    fwd = direction(hsf_ref[...], fcf_ref[...])
    bwd = direction(hsb_ref[...], fcb_ref[...])
    state = jnp.concatenate([fwd, bwd], axis=1)
    state = jax.nn.relu(
        state + jax.nn.relu(state @ R1_ref[...] + R1b_ref[...]) @ R2_ref[...]
        + R2b_ref[...])

    oh = (gnn_ref[...] == lax.broadcasted_iota(jnp.int32, (BLK2, G), 1))
    contrib = lax.dot_general(oh.astype(jnp.float32), state,
                              (((0,), (0,)), ((), ())))

    @pl.when(step == 0)
    def _():
        outg_ref[...] = contrib

    @pl.when(step != 0)
    def _():
        outg_ref[...] = outg_ref[...] + contrib

    @pl.when(step == nsteps - 1)
    def _():
        oh2 = (bgi_ref[...] == lax.broadcasted_iota(jnp.int32, (G, B), 1))
        obj = lax.dot_general(oh2.astype(jnp.float32), outg_ref[...],
                              (((0,), (0,)), ((), ())))
        out = jnp.concatenate([obj, jnp.zeros_like(obj)], axis=1)
        vf = jax.nn.relu(out @ V1_ref[...] + V1b_ref[...]) @ V2_ref[...] \
            + V2b_ref[...]
        lemma_q = jax.nn.relu(
            out + jax.nn.relu(out @ Q1_ref[...] + Q1b_ref[...]) @ Q2_ref[...]
            + Q2b_ref[...])
        lemma = jax.nn.relu(lemma_q) @ L_ref[...] + Lb_ref[...]
        out_ref[...] = jnp.concatenate([vf, lemma], axis=1)


def _run_k2(x_ioub, hsf, fcf, hsb, fcb, U_iou, W_out, R1, R1b, R2, R2b,
            gnn2d, bgi2d, V1, V1b, V2, V2b, Q1, Q1b, Q2, Q2b, L, Lb):
    n_blocks = N // BLK2
    full = lambda shape: pl.BlockSpec(shape, lambda i: (0,) * len(shape))
    row = lambda w: pl.BlockSpec((BLK2, w), lambda i: (i, 0))
    return pl.pallas_call(
        _k2_body,
        grid=(n_blocks,),
        in_specs=[
            row(3 * H), row(H), row(H), row(H), row(H),
            full((H, 3 * H)),
            full((H, 64)),
            full((2 * 64, 2 * 64)), full((1, 2 * 64)),
            full((2 * 64, 2 * 64)), full((1, 2 * 64)),
            pl.BlockSpec((BLK2, 1), lambda i: (i, 0)),
            full((G, 1)),
            full((2 * H, H)), full((1, H)),
            full((H, 1)), full((1, 1)),
            full((2 * H, 2 * H)), full((1, 2 * H)),
            full((2 * H, 2 * H)), full((1, 2 * H)),
            full((2 * H, NUM_LEMMAS)), full((1, NUM_LEMMAS)),
        ],
        out_specs=pl.BlockSpec((B, 1 + NUM_LEMMAS), lambda i: (0, 0)),
        out_shape=jax.ShapeDtypeStruct((B, 1 + NUM_LEMMAS), jnp.float32),
        scratch_shapes=[pltpu.VMEM((G, H), jnp.float32)],
    )(x_ioub, hsf, fcf, hsb, fcb, U_iou, W_out, R1, R1b, R2, R2b, gnn2d,
      bgi2d, V1, V1b, V2, V2b, Q1, Q1b, Q2, Q2b, L, Lb)


def kernel(node_ids, edge_src, edge_dst, gnn_ind, batch_gnn_ind, W_emb,
           W_iou, U_iou, b_iou, W_f, U_f, b_f, W_out, R1, R1b, R2, R2b, V1,
           V1b, V2, V2b, Q1, Q1b, Q2, Q2b, L, Lb):
    i32 = jnp.int32
    ids2d = node_ids.astype(i32).reshape(N, 1)
    gnn2d = gnn_ind.astype(i32).reshape(N, 1)
    bgi2d = batch_gnn_ind.astype(i32).reshape(G, 1)
    row1 = lambda v: v.reshape(1, -1)

    x_ioub, h1, pack, mxfb = _run_k1(
        ids2d, W_emb, W_iou, row1(b_iou), W_f, row1(b_f), U_f)

    gidx_all = jnp.concatenate([edge_src, edge_dst]).astype(i32)
    sidx_all = jnp.concatenate([edge_dst, edge_src]).astype(i32)
    hsum2, fc2 = _run_edge_sc(gidx_all, sidx_all, h1, pack, mxfb)
    hsf, fcf = hsum2[0], fc2[0]
    hsb, fcb = hsum2[1], fc2[1]

    return _run_k2(x_ioub, hsf, fcf, hsb, fcb, U_iou, W_out, R1, row1(R1b),
                   R2, row1(R2b), gnn2d, bgi2d, V1, row1(V1b), V2, row1(V2b),
                   Q1, row1(Q1b), Q2, row1(Q2b), L, row1(Lb))
